# baseline (speedup 1.0000x reference)
import functools

import jax
import jax.numpy as jnp
from jax import lax
from jax.experimental import pallas as pl
from jax.experimental.pallas import tpu as pltpu
from jax.experimental.pallas import tpu_sc as plsc

NUM_CORES = 2
NUM_SUBCORES = 16
NUM_WORKERS = NUM_CORES * NUM_SUBCORES
LANES = 16

SPLIT_A = 128
SPLIT_B = 72


def _make_pool_kernel(B, L, D):
  spt = B // NUM_WORKERS
  assert B % (8 * NUM_WORKERS) == 0 and L == SPLIT_A + SPLIT_B
  n_vec = D // LANES
  half = spt // 2

  mesh = plsc.VectorSubcoreMesh(
      core_axis_name="c", subcore_axis_name="s",
      num_cores=NUM_CORES, num_subcores=NUM_SUBCORES)

  @functools.partial(
      pl.kernel,
      out_type=jax.ShapeDtypeStruct((B * D,), jnp.float32),
      mesh=mesh,
      compiler_params=pltpu.CompilerParams(
          use_tc_tiling_on_sc=False, needs_layout_passes=False),
      scratch_types=[
          pltpu.VMEM((half * L,), jnp.int32),
          pltpu.VMEM((L, D), jnp.bfloat16),
          pltpu.VMEM((L, D), jnp.bfloat16),
          pltpu.VMEM((L, D), jnp.bfloat16),
          pltpu.VMEM((L, D), jnp.bfloat16),
          pltpu.VMEM((spt * D,), jnp.float32),
          pltpu.SemaphoreType.DMA,
          pltpu.SemaphoreType.DMA,
          pltpu.SemaphoreType.DMA,
          pltpu.SemaphoreType.DMA,
      ],
  )
  def pool(x_hbm, table_hbm, out_hbm, idx_v, rows0, rows1, rows2, rows3,
           out_v, sem0, sem1, sem2, sem3):
    wid = lax.axis_index("s") * NUM_CORES + lax.axis_index("c")
    base = wid * spt

    def issue_fetch(i, rows, sem):
      pltpu.async_copy(table_hbm.at[idx_v.at[pl.ds(i * L, SPLIT_A)]],
                       rows.at[pl.ds(0, SPLIT_A)], sem)
      pltpu.async_copy(table_hbm.at[idx_v.at[pl.ds(i * L + SPLIT_A, SPLIT_B)]],
                       rows.at[pl.ds(SPLIT_A, SPLIT_B)], sem)

    def wait_fetch(rows, sem):
      pltpu.make_async_copy(table_hbm.at[pl.ds(0, L)], rows, sem).wait()

    hi_mask = jnp.full((LANES,), -65536, jnp.int32)

    def accumulate(rows, i):
      def acc_body(t, accs):
        new = []
        for rr in range(2):
          r = 2 * t + rr
          for k in range(n_vec // 2):
            w = plsc.bitcast(rows[r, pl.ds(2 * LANES * k, 2 * LANES)],
                             jnp.int32)
            ev = plsc.bitcast(lax.shift_left(w, 16), jnp.float32)
            od = plsc.bitcast(jnp.bitwise_and(w, hi_mask), jnp.float32)
            j = rr * n_vec + 2 * k
            new.append(accs[j] + ev)
            new.append(accs[j + 1] + od)
        return tuple(new)

      init = tuple(jnp.zeros((LANES,), jnp.float32) for _ in range(2 * n_vec))
      accs = lax.fori_loop(0, L // 2, acc_body, init, unroll=10)
      for k in range(n_vec):
        out_v[pl.ds(i * D + LANES * k, LANES)] = accs[k] + accs[k + n_vec]

    bufs = ((rows0, sem0), (rows1, sem1), (rows2, sem2), (rows3, sem3))

    for h in range(2):
      pltpu.sync_copy(x_hbm.at[pl.ds((base + h * half) * L, half * L)], idx_v)
      for t in range(3):
        issue_fetch(t, bufs[t][0], bufs[t][1])

      def body(j, _, h=h):
        i = 4 * j
        for t in range(4):
          nxt = i + t + 3

          @pl.when(nxt < half)
          def _(nxt=nxt, t=t):
            issue_fetch(nxt, bufs[(t + 3) % 4][0], bufs[(t + 3) % 4][1])

          wait_fetch(bufs[t][0], bufs[t][1])
          accumulate(bufs[t][0], h * half + i + t)
        return 0

      lax.fori_loop(0, half // 4, body, 0)

    pltpu.sync_copy(out_v, out_hbm.at[pl.ds(base * D, spt * D)])

  return pool


def _mlp_body(pool_ref, len_ref, w1_ref, b1_ref, w2_ref, b2_ref, out_ref):
  rep = pool_ref[...] / len_ref[...]
  h = jnp.dot(rep, w1_ref[...], preferred_element_type=jnp.float32)
  h = jnp.maximum(h + b1_ref[...], 0.0)
  out = jnp.dot(h, w2_ref[...], preferred_element_type=jnp.float32)
  out_ref[...] = out + b2_ref[...]


def _mlp(pooled, lengths_f, w1, b1, w2, b2):
  B, D = pooled.shape
  hid = w1.shape[1]
  out_dim = w2.shape[1]
  blk = 2048
  grid = (B // blk,)
  return pl.pallas_call(
      _mlp_body,
      grid=grid,
      in_specs=[
          pl.BlockSpec((blk, D), lambda i: (i, 0)),
          pl.BlockSpec((blk, 1), lambda i: (i, 0)),
          pl.BlockSpec((D, hid), lambda i: (0, 0)),
          pl.BlockSpec((1, hid), lambda i: (0, 0)),
          pl.BlockSpec((hid, out_dim), lambda i: (0, 0)),
          pl.BlockSpec((1, out_dim), lambda i: (0, 0)),
      ],
      out_specs=pl.BlockSpec((blk, out_dim), lambda i: (i, 0)),
      out_shape=jax.ShapeDtypeStruct((B, out_dim), jnp.float32),
  )(pooled, lengths_f, w1, b1, w2, b2)


def _unpack_perm(D):
  half = D // 2
  return (list(range(0, half, 2)) + list(range(1, half, 2)) +
          list(range(half, D, 2)) + list(range(half + 1, D, 2)))


def kernel(x, lengths, emb_table, W1, b1, W2, b2):
  B, L = x.shape
  D = emb_table.shape[1]
  pooled = _make_pool_kernel(B, L, D)(
      x.reshape(B * L), emb_table.astype(jnp.bfloat16))
  pooled = pooled.reshape(B, D)
  lengths_f = lengths.astype(jnp.float32).reshape(B, 1)
  w1p = W1[jnp.array(_unpack_perm(D), dtype=jnp.int32), :]
  return _mlp(pooled, lengths_f, w1p, b1.reshape(1, -1), W2, b2.reshape(1, -1))


# speedup vs baseline: 1.0256x; 1.0256x over previous
import functools

import jax
import jax.numpy as jnp
from jax import lax
from jax.experimental import pallas as pl
from jax.experimental.pallas import tpu as pltpu
from jax.experimental.pallas import tpu_sc as plsc

NUM_CORES = 2
NUM_SUBCORES = 16
NUM_WORKERS = NUM_CORES * NUM_SUBCORES
LANES = 16

SPLIT_A = 128
SPLIT_B = 72


def _make_pool_kernel(B, L, D):
  spt = B // NUM_WORKERS
  assert B % (8 * NUM_WORKERS) == 0 and L == SPLIT_A + SPLIT_B
  n_vec = D // LANES
  half = spt // 2

  mesh = plsc.VectorSubcoreMesh(
      core_axis_name="c", subcore_axis_name="s",
      num_cores=NUM_CORES, num_subcores=NUM_SUBCORES)

  @functools.partial(
      pl.kernel,
      out_type=jax.ShapeDtypeStruct((B * D,), jnp.float32),
      mesh=mesh,
      compiler_params=pltpu.CompilerParams(
          use_tc_tiling_on_sc=False, needs_layout_passes=False),
      scratch_types=(
          [pltpu.VMEM((half * L,), jnp.int32)]
          + [pltpu.VMEM((L, D), jnp.bfloat16) for _ in range(8)]
          + [pltpu.VMEM((half * D,), jnp.float32)]
          + [pltpu.SemaphoreType.DMA for _ in range(8)]
      ),
  )
  def pool(x_hbm, table_hbm, out_hbm, idx_v, *rest):
    row_bufs = rest[:8]
    out_v = rest[8]
    sems = rest[9:17]
    wid = lax.axis_index("s") * NUM_CORES + lax.axis_index("c")
    base = wid * spt

    def issue_fetch(i, rows, sem):
      pltpu.async_copy(table_hbm.at[idx_v.at[pl.ds(i * L, SPLIT_A)]],
                       rows.at[pl.ds(0, SPLIT_A)], sem)
      pltpu.async_copy(table_hbm.at[idx_v.at[pl.ds(i * L + SPLIT_A, SPLIT_B)]],
                       rows.at[pl.ds(SPLIT_A, SPLIT_B)], sem)

    def wait_fetch(rows, sem):
      pltpu.make_async_copy(table_hbm.at[pl.ds(0, L)], rows, sem).wait()

    hi_mask = jnp.full((LANES,), -65536, jnp.int32)

    def accumulate(rows, i):
      def acc_body(t, accs):
        new = []
        for rr in range(2):
          r = 2 * t + rr
          for k in range(n_vec // 2):
            w = plsc.bitcast(rows[r, pl.ds(2 * LANES * k, 2 * LANES)],
                             jnp.int32)
            ev = plsc.bitcast(lax.shift_left(w, 16), jnp.float32)
            od = plsc.bitcast(jnp.bitwise_and(w, hi_mask), jnp.float32)
            j = rr * n_vec + 2 * k
            new.append(accs[j] + ev)
            new.append(accs[j + 1] + od)
        return tuple(new)

      init = tuple(jnp.zeros((LANES,), jnp.float32) for _ in range(2 * n_vec))
      accs = lax.fori_loop(0, L // 2, acc_body, init, unroll=10)
      for k in range(n_vec):
        out_v[pl.ds(i * D + LANES * k, LANES)] = accs[k] + accs[k + n_vec]

    bufs = tuple(zip(row_bufs, sems))

    for h in range(2):
      pltpu.sync_copy(x_hbm.at[pl.ds((base + h * half) * L, half * L)], idx_v)
      for t in range(7):
        issue_fetch(t, bufs[t][0], bufs[t][1])

      def body(j, _):
        i = 8 * j
        for t in range(8):
          nxt = i + t + 7

          @pl.when(nxt < half)
          def _(nxt=nxt, t=t):
            issue_fetch(nxt, bufs[(t + 7) % 8][0], bufs[(t + 7) % 8][1])

          wait_fetch(bufs[t][0], bufs[t][1])
          accumulate(bufs[t][0], i + t)
        return 0

      lax.fori_loop(0, half // 8, body, 0)
      pltpu.sync_copy(out_v, out_hbm.at[pl.ds((base + h * half) * D, half * D)])

  return pool


def _mlp_body(pool_ref, len_ref, w1_ref, b1_ref, w2_ref, b2_ref, out_ref):
  rep = pool_ref[...] / len_ref[...]
  h = jnp.dot(rep, w1_ref[...], preferred_element_type=jnp.float32)
  h = jnp.maximum(h + b1_ref[...], 0.0)
  out = jnp.dot(h, w2_ref[...], preferred_element_type=jnp.float32)
  out_ref[...] = out + b2_ref[...]


def _mlp(pooled, lengths_f, w1, b1, w2, b2):
  B, D = pooled.shape
  hid = w1.shape[1]
  out_dim = w2.shape[1]
  blk = 2048
  grid = (B // blk,)
  return pl.pallas_call(
      _mlp_body,
      grid=grid,
      in_specs=[
          pl.BlockSpec((blk, D), lambda i: (i, 0)),
          pl.BlockSpec((blk, 1), lambda i: (i, 0)),
          pl.BlockSpec((D, hid), lambda i: (0, 0)),
          pl.BlockSpec((1, hid), lambda i: (0, 0)),
          pl.BlockSpec((hid, out_dim), lambda i: (0, 0)),
          pl.BlockSpec((1, out_dim), lambda i: (0, 0)),
      ],
      out_specs=pl.BlockSpec((blk, out_dim), lambda i: (i, 0)),
      out_shape=jax.ShapeDtypeStruct((B, out_dim), jnp.float32),
  )(pooled, lengths_f, w1, b1, w2, b2)


def _unpack_perm(D):
  half = D // 2
  return (list(range(0, half, 2)) + list(range(1, half, 2)) +
          list(range(half, D, 2)) + list(range(half + 1, D, 2)))


def kernel(x, lengths, emb_table, W1, b1, W2, b2):
  B, L = x.shape
  D = emb_table.shape[1]
  pooled = _make_pool_kernel(B, L, D)(
      x.reshape(B * L), emb_table.astype(jnp.bfloat16))
  pooled = pooled.reshape(B, D)
  lengths_f = lengths.astype(jnp.float32).reshape(B, 1)
  w1p = W1[jnp.array(_unpack_perm(D), dtype=jnp.int32), :]
  return _mlp(pooled, lengths_f, w1p, b1.reshape(1, -1), W2, b2.reshape(1, -1))


# speedup vs baseline: 1.0431x; 1.0170x over previous
import functools

import jax
import jax.numpy as jnp
from jax import lax
from jax.experimental import pallas as pl
from jax.experimental.pallas import tpu as pltpu
from jax.experimental.pallas import tpu_sc as plsc

NUM_CORES = 2
NUM_SUBCORES = 16
NUM_WORKERS = NUM_CORES * NUM_SUBCORES
LANES = 16

SPLIT_A = 128
SPLIT_B = 72


def _make_pool_kernel(B, L, D):
  spt = B // NUM_WORKERS
  assert B % (8 * NUM_WORKERS) == 0 and L == SPLIT_A + SPLIT_B
  n_vec = D // LANES
  half = spt // 2

  mesh = plsc.VectorSubcoreMesh(
      core_axis_name="c", subcore_axis_name="s",
      num_cores=NUM_CORES, num_subcores=NUM_SUBCORES)

  @functools.partial(
      pl.kernel,
      out_type=jax.ShapeDtypeStruct((B * D,), jnp.float32),
      mesh=mesh,
      compiler_params=pltpu.CompilerParams(
          use_tc_tiling_on_sc=False, needs_layout_passes=False),
      scratch_types=(
          [pltpu.VMEM((half * L,), jnp.int32)]
          + [pltpu.VMEM((L, D), jnp.bfloat16) for _ in range(8)]
          + [pltpu.VMEM((half * D,), jnp.float32)]
          + [pltpu.SemaphoreType.DMA for _ in range(8)]
      ),
  )
  def pool(x_hbm, table_hbm, out_hbm, idx_v, *rest):
    row_bufs = rest[:8]
    out_v = rest[8]
    sems = rest[9:17]
    wid = lax.axis_index("s") * NUM_CORES + lax.axis_index("c")
    rpt = spt // 2
    row0 = wid * rpt

    def issue_fetch(i, rows, sem):
      pltpu.async_copy(table_hbm.at[idx_v.at[pl.ds(i * L, SPLIT_A)]],
                       rows.at[pl.ds(0, SPLIT_A)], sem)
      pltpu.async_copy(table_hbm.at[idx_v.at[pl.ds(i * L + SPLIT_A, SPLIT_B)]],
                       rows.at[pl.ds(SPLIT_A, SPLIT_B)], sem)

    def wait_fetch(rows, sem):
      pltpu.make_async_copy(table_hbm.at[pl.ds(0, L)], rows, sem).wait()

    hi_mask = jnp.full((LANES,), -65536, jnp.int32)

    def accumulate(rows, i):
      def acc_body(t, accs):
        new = []
        for rr in range(2):
          r = 2 * t + rr
          for k in range(n_vec // 2):
            w = plsc.bitcast(rows[r, pl.ds(2 * LANES * k, 2 * LANES)],
                             jnp.int32)
            ev = plsc.bitcast(lax.shift_left(w, 16), jnp.float32)
            od = plsc.bitcast(jnp.bitwise_and(w, hi_mask), jnp.float32)
            j = rr * n_vec + 2 * k
            new.append(accs[j] + ev)
            new.append(accs[j + 1] + od)
        return tuple(new)

      init = tuple(jnp.zeros((LANES,), jnp.float32) for _ in range(2 * n_vec))
      accs = lax.fori_loop(0, L // 2, acc_body, init, unroll=10)
      hr = half // 2
      off = jnp.where(i < hr, i * 2 * D, (i - hr) * 2 * D + D)
      for k in range(n_vec):
        out_v[pl.ds(off + LANES * k, LANES)] = accs[k] + accs[k + n_vec]

    bufs = tuple(zip(row_bufs, sems))

    hr = half // 2
    for h in range(2):
      pltpu.sync_copy(x_hbm.at[pl.ds((row0 + h * hr) * L, hr * L)],
                      idx_v.at[pl.ds(0, hr * L)])
      pltpu.sync_copy(x_hbm.at[pl.ds((B // 2 + row0 + h * hr) * L, hr * L)],
                      idx_v.at[pl.ds(hr * L, hr * L)])
      for t in range(7):
        issue_fetch(t, bufs[t][0], bufs[t][1])

      def body(j, _):
        i = 8 * j
        for t in range(8):
          nxt = i + t + 7

          @pl.when(nxt < half)
          def _(nxt=nxt, t=t):
            issue_fetch(nxt, bufs[(t + 7) % 8][0], bufs[(t + 7) % 8][1])

          wait_fetch(bufs[t][0], bufs[t][1])
          accumulate(bufs[t][0], i + t)
        return 0

      lax.fori_loop(0, half // 8, body, 0)
      pltpu.sync_copy(out_v,
                      out_hbm.at[pl.ds((row0 + h * hr) * 2 * D, half * D)])

  return pool


def _mlp_body(pool_ref, len_ref, w1_ref, b1_ref, w2_ref, b2_ref, out_ref):
  j = pl.program_id(0)
  full = pool_ref[...]
  half_d = full.shape[1] // 2
  rep = jnp.where(j == 0, full[:, :half_d], full[:, half_d:])
  rep = rep / len_ref[...]
  h = jnp.dot(rep, w1_ref[...], preferred_element_type=jnp.float32)
  h = jnp.maximum(h + b1_ref[...], 0.0)
  out = jnp.dot(h, w2_ref[...], preferred_element_type=jnp.float32)
  out_ref[...] = out + b2_ref[...]


def _mlp(pooled2, lengths_f, w1, b1, w2, b2):
  B2, D2 = pooled2.shape
  D = D2 // 2
  hid = w1.shape[1]
  out_dim = w2.shape[1]
  blk = 2048
  nb = B2 // blk
  return pl.pallas_call(
      _mlp_body,
      grid=(2, nb),
      in_specs=[
          pl.BlockSpec((blk, D2), lambda j, i: (i, 0)),
          pl.BlockSpec((blk, 1), lambda j, i: (j * nb + i, 0)),
          pl.BlockSpec((D, hid), lambda j, i: (0, 0)),
          pl.BlockSpec((1, hid), lambda j, i: (0, 0)),
          pl.BlockSpec((hid, out_dim), lambda j, i: (0, 0)),
          pl.BlockSpec((1, out_dim), lambda j, i: (0, 0)),
      ],
      out_specs=pl.BlockSpec((blk, out_dim), lambda j, i: (j * nb + i, 0)),
      out_shape=jax.ShapeDtypeStruct((2 * B2, out_dim), jnp.float32),
  )(pooled2, lengths_f, w1, b1, w2, b2)


def _unpack_perm(D):
  half = D // 2
  return (list(range(0, half, 2)) + list(range(1, half, 2)) +
          list(range(half, D, 2)) + list(range(half + 1, D, 2)))


def kernel(x, lengths, emb_table, W1, b1, W2, b2):
  B, L = x.shape
  D = emb_table.shape[1]
  pooled = _make_pool_kernel(B, L, D)(
      x.reshape(B * L), emb_table.astype(jnp.bfloat16))
  pooled2 = pooled.reshape(B // 2, 2 * D)
  lengths_f = lengths.astype(jnp.float32).reshape(B, 1)
  w1p = W1[jnp.array(_unpack_perm(D), dtype=jnp.int32), :]
  return _mlp(pooled2, lengths_f, w1p, b1.reshape(1, -1), W2, b2.reshape(1, -1))
